# Initial kernel scaffold; baseline (speedup 1.0000x reference)
import jax
import jax.numpy as jnp
from jax.experimental import pallas as pl


def kernel(x, embedding):
    raise NotImplementedError("write your pallas kernel here")


# speedup vs baseline: 1.0946x; 1.0946x over previous
import functools

import jax
import jax.numpy as jnp
from jax import lax
from jax.experimental import pallas as pl
from jax.experimental.pallas import tpu as pltpu
from jax.experimental.pallas import tpu_sc as plsc

_VOCAB = 1000000
_DEPTH = 32
_B = 16384
_L = 50
_NTOT = _B * _L

_info = plsc.get_sparse_core_info()
_NC = _info.num_cores
_NS = _info.num_subcores
_NW = _NC * _NS
_TRANS = 128
_M = 8
_CHUNK = _TRANS * _M
_PER_W = _NTOT // _NW
_NCHUNK = _PER_W // _CHUNK
_XROWS_PER_W = _PER_W // _TRANS

_mesh = plsc.VectorSubcoreMesh(core_axis_name="c", subcore_axis_name="s")


@functools.partial(
    pl.kernel,
    mesh=_mesh,
    out_type=jax.ShapeDtypeStruct((_NTOT, _DEPTH), jnp.int32),
    scratch_types=[
        pltpu.VMEM((_M, _TRANS), jnp.int32),
        pltpu.VMEM((_CHUNK, _DEPTH), jnp.int32),
        pltpu.SemaphoreType.DMA,
    ],
    compiler_params=pltpu.CompilerParams(use_tc_tiling_on_sc=False),
)
def _gather_kernel(x_hbm, emb_hbm, out_hbm, idx_v, rows_v, sem):
    wid = lax.axis_index("s") * _NC + lax.axis_index("c")
    xrow_base = wid * _XROWS_PER_W
    out_base = wid * _PER_W

    def chunk_body(c, carry):
        pltpu.sync_copy(x_hbm.at[pl.ds(xrow_base + c * _M, _M)], idx_v)
        copies = []
        for j in range(_M):
            copies.append(
                pltpu.async_copy(
                    emb_hbm.at[idx_v.at[j]],
                    rows_v.at[pl.ds(j * _TRANS, _TRANS)],
                    sem,
                )
            )
        for cp in copies:
            cp.wait()
        pltpu.sync_copy(rows_v, out_hbm.at[pl.ds(out_base + c * _CHUNK, _CHUNK)])
        return carry

    lax.fori_loop(0, _NCHUNK, chunk_body, 0)


def kernel(x, embedding):
    x_rows = x.reshape(_NTOT // _TRANS, _TRANS)
    out = _gather_kernel(x_rows, embedding)
    return out.reshape(_B, _L, _DEPTH)


# speedup vs baseline: 1.1140x; 1.0178x over previous
import functools

import jax
import jax.numpy as jnp
from jax import lax
from jax.experimental import pallas as pl
from jax.experimental.pallas import tpu as pltpu
from jax.experimental.pallas import tpu_sc as plsc

_VOCAB = 1000000
_DEPTH = 32
_B = 16384
_L = 50
_NTOT = _B * _L

_info = plsc.get_sparse_core_info()
_NC = _info.num_cores
_NS = _info.num_subcores
_NW = _NC * _NS
_TRANS = 128
_M = 10
_CHUNK = _TRANS * _M
_PER_W = _NTOT // _NW
_NCHUNK = _PER_W // _CHUNK
_NPAIR = _NCHUNK // 2
_XROWS_PER_W = _PER_W // _TRANS

_mesh = plsc.VectorSubcoreMesh(core_axis_name="c", subcore_axis_name="s")


@functools.partial(
    pl.kernel,
    mesh=_mesh,
    out_type=jax.ShapeDtypeStruct((_NTOT, _DEPTH), jnp.int32),
    scratch_types=[
        pltpu.VMEM((_XROWS_PER_W, _TRANS), jnp.int32),
        pltpu.VMEM((_CHUNK, _DEPTH), jnp.int32),
        pltpu.VMEM((_CHUNK, _DEPTH), jnp.int32),
        pltpu.SemaphoreType.DMA,
        pltpu.SemaphoreType.DMA,
        pltpu.SemaphoreType.DMA,
        pltpu.SemaphoreType.DMA,
    ],
    compiler_params=pltpu.CompilerParams(use_tc_tiling_on_sc=False),
)
def _gather_kernel(x_hbm, emb_hbm, out_hbm, idx_v, rows0, rows1,
                   gsem0, gsem1, wsem0, wsem1):
    wid = lax.axis_index("s") * _NC + lax.axis_index("c")
    out_base = wid * _PER_W

    pltpu.sync_copy(x_hbm.at[pl.ds(wid * _XROWS_PER_W, _XROWS_PER_W)], idx_v)

    rows = (rows0, rows1)
    gsems = (gsem0, gsem1)
    wsems = (wsem0, wsem1)

    def fire(c, b):
        for j in range(_M):
            pltpu.async_copy(
                emb_hbm.at[idx_v.at[c * _M + j]],
                rows[b].at[pl.ds(j * _TRANS, _TRANS)],
                gsems[b],
            )

    def drain_and_writeback(c, b):
        pltpu.make_async_copy(
            out_hbm.at[pl.ds(0, _CHUNK)], rows[b], gsems[b]
        ).wait()
        pltpu.async_copy(
            rows[b], out_hbm.at[pl.ds(out_base + c * _CHUNK, _CHUNK)], wsems[b]
        )

    def wait_writeback(b):
        pltpu.make_async_copy(
            out_hbm.at[pl.ds(0, _CHUNK)], rows[b], wsems[b]
        ).wait()

    fire(0, 0)

    def pair_body(i, carry):
        c0 = 2 * i
        c1 = c0 + 1

        @pl.when(i > 0)
        def _():
            wait_writeback(1)
        fire(c1, 1)
        drain_and_writeback(c0, 0)

        @pl.when(i < _NPAIR - 1)
        def _():
            wait_writeback(0)
            fire(c0 + 2, 0)
        drain_and_writeback(c1, 1)
        return carry

    lax.fori_loop(0, _NPAIR, pair_body, 0)
    wait_writeback(0)
    wait_writeback(1)


def kernel(x, embedding):
    x_rows = x.reshape(_NTOT // _TRANS, _TRANS)
    out = _gather_kernel(x_rows, embedding)
    return out.reshape(_B, _L, _DEPTH)


# speedup vs baseline: 1.8067x; 1.6218x over previous
import functools

import jax
import jax.numpy as jnp
from jax import lax
from jax.experimental import pallas as pl
from jax.experimental.pallas import tpu as pltpu
from jax.experimental.pallas import tpu_sc as plsc

_VOCAB = 1000000
_DEPTH = 32
_B = 16384
_L = 50

_info = plsc.get_sparse_core_info()
_NC = _info.num_cores
_NS = _info.num_subcores
_NW = _NC * _NS
_B_PER_W = _B // _NW
_CB = 16
_NCHUNK = _B_PER_W // _CB
_NPAIR = _NCHUNK // 2

_mesh = plsc.VectorSubcoreMesh(core_axis_name="c", subcore_axis_name="s")


@functools.partial(
    pl.kernel,
    mesh=_mesh,
    out_type=jax.ShapeDtypeStruct((_B, _L, _DEPTH), jnp.int32),
    scratch_types=[
        pltpu.VMEM((_B_PER_W, _L), jnp.int32),
        pltpu.VMEM((_CB, _L, _DEPTH), jnp.int32),
        pltpu.VMEM((_CB, _L, _DEPTH), jnp.int32),
        pltpu.SemaphoreType.DMA,
        pltpu.SemaphoreType.DMA,
        pltpu.SemaphoreType.DMA,
        pltpu.SemaphoreType.DMA,
    ],
    compiler_params=pltpu.CompilerParams(use_tc_tiling_on_sc=False),
)
def _gather_kernel(x_hbm, emb_hbm, out_hbm, idx_v, rows0, rows1,
                   gsem0, gsem1, wsem0, wsem1):
    wid = lax.axis_index("s") * _NC + lax.axis_index("c")
    b_base = wid * _B_PER_W

    pltpu.sync_copy(x_hbm.at[pl.ds(b_base, _B_PER_W)], idx_v)

    rows = (rows0, rows1)
    gsems = (gsem0, gsem1)
    wsems = (wsem0, wsem1)

    def fire(c, b):
        for j in range(_CB):
            pltpu.async_copy(
                emb_hbm.at[idx_v.at[c * _CB + j]],
                rows[b].at[j],
                gsems[b],
            )

    def drain_and_writeback(c, b):
        pltpu.make_async_copy(
            out_hbm.at[pl.ds(0, _CB)], rows[b], gsems[b]
        ).wait()
        pltpu.async_copy(
            rows[b], out_hbm.at[pl.ds(b_base + c * _CB, _CB)], wsems[b]
        )

    def wait_writeback(b):
        pltpu.make_async_copy(
            out_hbm.at[pl.ds(0, _CB)], rows[b], wsems[b]
        ).wait()

    fire(0, 0)

    def pair_body(i, carry):
        c0 = 2 * i
        c1 = c0 + 1

        @pl.when(i > 0)
        def _():
            wait_writeback(1)
        fire(c1, 1)
        drain_and_writeback(c0, 0)

        @pl.when(i < _NPAIR - 1)
        def _():
            wait_writeback(0)
            fire(c0 + 2, 0)
        drain_and_writeback(c1, 1)
        return carry

    lax.fori_loop(0, _NPAIR, pair_body, 0)
    wait_writeback(0)
    wait_writeback(1)


def kernel(x, embedding):
    return _gather_kernel(x, embedding)
